# Initial kernel scaffold; baseline (speedup 1.0000x reference)
import jax
import jax.numpy as jnp
from jax.experimental import pallas as pl


def kernel(context, target_log, mask, spatial_ei, transit_ei, W_c1, b_c1, g_c, bb_c, W_c2, b_c2, mask_token, W_t, b_t, W_s1, b_s1, W_s2, b_s2, W_r1, b_r1, W_r2, b_r2, alpha, W_p1, b_p1, g_p, bb_p, W_p2, b_p2, W_p3, b_p3):
    raise NotImplementedError("write your pallas kernel here")


# speedup vs baseline: 3.4011x; 3.4011x over previous
import functools
import math

import jax
import jax.numpy as jnp
from jax import lax
from jax.experimental import pallas as pl
from jax.experimental.pallas import tpu as pltpu
from jax.experimental.pallas import tpu_sc as plsc

N = 50000
E = 800000

_NC = 2
_NS = 16
_SUB = 128
_HALF_E = E // _NC
_CHUNKS_PER_TILE = -(-_HALF_E // (_SUB * _NS))
_CHUNKS_PER_CORE = _CHUNKS_PER_TILE * _NS
_PAD_E = _CHUNKS_PER_CORE * _SUB
_NTRASH = 176
_NACC = N + _NTRASH
_ROWS_PER_TILE = _NACC // _NS

_SROWS = _ROWS_PER_TILE // 8

_BLK = 2000
_GRID = N // _BLK


def _pack_edges(ei):
    row = ei[0].astype(jnp.int32).reshape(_NC, _HALF_E)
    col = ei[1].astype(jnp.int32).reshape(_NC, _HALF_E)
    pad = _PAD_E - _HALF_E
    trash = (N + (jnp.arange(pad, dtype=jnp.int32) % _NTRASH))[None, :]
    rowp = jnp.concatenate([row, jnp.broadcast_to(trash, (_NC, pad))], axis=1)
    colp = jnp.concatenate([col, jnp.zeros((_NC, pad), jnp.int32)], axis=1)
    packed = jnp.stack(
        [rowp.reshape(_NC, _CHUNKS_PER_CORE, _SUB),
         colp.reshape(_NC, _CHUNKS_PER_CORE, _SUB)], axis=2)
    return packed.reshape(_NC * _CHUNKS_PER_CORE, 2, _SUB)


def _combo(e_ref, y_ref, out_ref, zeros2d, acc, idx_v, rows_v, stage_v,
           sem, c, s):
    r0 = s * _ROWS_PER_TILE
    pltpu.sync_copy(zeros2d, stage_v)
    for h in range(8):
        pltpu.sync_copy(stage_v, acc.at[pl.ds(r0 + h * _SROWS, _SROWS)])
    plsc.subcore_barrier()
    base = c * _CHUNKS_PER_CORE + s * _CHUNKS_PER_TILE

    def body(j, _):
        chunk = base + j
        pltpu.sync_copy(e_ref.at[chunk], idx_v)
        pltpu.async_copy(y_ref.at[idx_v.at[1]], rows_v, sem).wait()
        pltpu.sync_copy(rows_v, acc.at[idx_v.at[0]], add=True)
        return 0

    lax.fori_loop(0, _CHUNKS_PER_TILE, body, 0)
    plsc.subcore_barrier()
    for h in range(8):
        pltpu.sync_copy(acc.at[pl.ds(r0 + h * _SROWS, _SROWS)], stage_v)
        pltpu.sync_copy(
            stage_v, out_ref.at[pl.ds(c * _NACC + r0 + h * _SROWS, _SROWS)])


def _sc_prop1(es, et, ysl, ysh, yrl, yrh, zeros2d, zeros1d, ones128,
              degs_p, degr_p, psl, psh, prl, prh,
              acc, dacc_s, dacc_r, idx_v, rows_v, stage_v, dstage_v,
              ones_v, sem):
    c = lax.axis_index("c")
    s = lax.axis_index("s")
    pltpu.sync_copy(ones128, ones_v)
    r0 = s * _ROWS_PER_TILE
    pltpu.sync_copy(zeros1d, dstage_v)
    pltpu.sync_copy(dstage_v, dacc_s.at[pl.ds(r0, _ROWS_PER_TILE)])
    pltpu.sync_copy(dstage_v, dacc_r.at[pl.ds(r0, _ROWS_PER_TILE)])
    plsc.subcore_barrier()
    base = c * _CHUNKS_PER_CORE + s * _CHUNKS_PER_TILE

    def dbody(j, _):
        chunk = base + j
        pltpu.sync_copy(es.at[chunk], idx_v)
        pltpu.sync_copy(ones_v, dacc_s.at[idx_v.at[0]], add=True)
        pltpu.sync_copy(et.at[chunk], idx_v)
        pltpu.sync_copy(ones_v, dacc_r.at[idx_v.at[0]], add=True)
        return 0

    lax.fori_loop(0, _CHUNKS_PER_TILE, dbody, 0)
    plsc.subcore_barrier()
    pltpu.sync_copy(dacc_s.at[pl.ds(r0, _ROWS_PER_TILE)], dstage_v)
    pltpu.sync_copy(dstage_v, degs_p.at[pl.ds(c * _NACC + r0, _ROWS_PER_TILE)])
    pltpu.sync_copy(dacc_r.at[pl.ds(r0, _ROWS_PER_TILE)], dstage_v)
    pltpu.sync_copy(dstage_v, degr_p.at[pl.ds(c * _NACC + r0, _ROWS_PER_TILE)])
    args = (acc, idx_v, rows_v, stage_v, sem, c, s)
    _combo(es, ysl, psl, zeros2d, *args)
    _combo(es, ysh, psh, zeros2d, *args)
    _combo(et, yrl, prl, zeros2d, *args)
    _combo(et, yrh, prh, zeros2d, *args)


def _sc_prop2(es, et, zsl, zsh, zrl, zrh, zeros2d,
              qsl, qsh, qrl, qrh,
              acc, idx_v, rows_v, stage_v, sem):
    c = lax.axis_index("c")
    s = lax.axis_index("s")
    args = (acc, idx_v, rows_v, stage_v, sem, c, s)
    _combo(es, zsl, qsl, zeros2d, *args)
    _combo(es, zsh, qsh, zeros2d, *args)
    _combo(et, zrl, qrl, zeros2d, *args)
    _combo(et, zrh, qrh, zeros2d, *args)


_part2d = jax.ShapeDtypeStruct((_NC * _NACC,), jnp.float32)
_part3d = jax.ShapeDtypeStruct((_NC * _NACC, 32), jnp.float32)
_sc_mesh = plsc.VectorSubcoreMesh(core_axis_name="c", subcore_axis_name="s")

_sc_params = pltpu.CompilerParams(use_tc_tiling_on_sc=False)

_prop1_call = pl.kernel(
    _sc_prop1,
    out_type=(_part2d, _part2d, _part3d, _part3d, _part3d, _part3d),
    mesh=_sc_mesh,
    compiler_params=_sc_params,
    scratch_types=[
        pltpu.VMEM_SHARED((_NACC, 32), jnp.float32),
        pltpu.VMEM_SHARED((_NACC,), jnp.float32),
        pltpu.VMEM_SHARED((_NACC,), jnp.float32),
        pltpu.VMEM((2, _SUB), jnp.int32),
        pltpu.VMEM((_SUB, 32), jnp.float32),
        pltpu.VMEM((_SROWS, 32), jnp.float32),
        pltpu.VMEM((_ROWS_PER_TILE,), jnp.float32),
        pltpu.VMEM((_SUB,), jnp.float32),
        pltpu.SemaphoreType.DMA,
    ],
)

_prop2_call = pl.kernel(
    _sc_prop2,
    out_type=(_part3d, _part3d, _part3d, _part3d),
    mesh=_sc_mesh,
    compiler_params=_sc_params,
    scratch_types=[
        pltpu.VMEM_SHARED((_NACC, 32), jnp.float32),
        pltpu.VMEM((2, _SUB), jnp.int32),
        pltpu.VMEM((_SUB, 32), jnp.float32),
        pltpu.VMEM((_SROWS, 32), jnp.float32),
        pltpu.SemaphoreType.DMA,
    ],
)


def _dot(a, b):
    return jnp.dot(a, b, preferred_element_type=jnp.float32)


def _enc_body(ctx_ref, tl_ref, m_ref, wc1, bc1, wc2, bc2, mtok, wt, bt,
              ws1, wr1, fused_ref, ysl_ref, ysh_ref, yrl_ref, yrh_ref):
    h = jax.nn.relu(_dot(ctx_ref[...], wc1[...]) + bc1[...])
    ctx = jax.nn.relu(_dot(h, wc2[...]) + bc2[...])
    m = m_ref[...]
    masked = tl_ref[...] * (1.0 - m) + mtok[...] * m
    tgt = jax.nn.relu(_dot(masked, wt[...]) + bt[...])
    fused = jnp.concatenate([ctx, tgt], axis=1)
    fused_ref[...] = fused
    ys = _dot(fused, ws1[...])
    yr = _dot(fused, wr1[...])
    ysl_ref[...] = ys[:, :32]
    ysh_ref[...] = ys[:, 32:]
    yrl_ref[...] = yr[:, :32]
    yrh_ref[...] = yr[:, 32:]


def _mid_body(psl, psh, prl, prh, dsp, drp, bs1, ws2, br1, wr2,
              zsl_ref, zsh_ref, zrl_ref, zrh_ref):
    rs = 1.0 / jnp.clip(dsp[0] + dsp[1], 1.0, None)
    rr = 1.0 / jnp.clip(drp[0] + drp[1], 1.0, None)
    aggs = jnp.concatenate([psl[0] + psl[1], psh[0] + psh[1]], axis=1)
    aggr = jnp.concatenate([prl[0] + prl[1], prh[0] + prh[1]], axis=1)
    h1s = jax.nn.relu(aggs * rs + bs1[...])
    h1r = jax.nn.relu(aggr * rr + br1[...])
    zs = _dot(h1s, ws2[...])
    zr = _dot(h1r, wr2[...])
    zsl_ref[...] = zs[:, :32]
    zsh_ref[...] = zs[:, 32:]
    zrl_ref[...] = zr[:, :32]
    zrh_ref[...] = zr[:, 32:]


def _head_body(qsl, qsh, qrl, qrh, dsp, drp, fused_ref, bs2, br2,
               wp1s, wp1t, wp1f, bp1, wp2, bp2, wp3, bp3, out_ref):
    rs = 1.0 / jnp.clip(dsp[0] + dsp[1], 1.0, None)
    rr = 1.0 / jnp.clip(drp[0] + drp[1], 1.0, None)
    hs2 = jnp.concatenate([qsl[0] + qsl[1], qsh[0] + qsh[1]], axis=1) * rs + bs2[...]
    ht2 = jnp.concatenate([qrl[0] + qrl[1], qrh[0] + qrh[1]], axis=1) * rr + br2[...]
    h = jax.nn.relu(_dot(hs2, wp1s[...]) + _dot(ht2, wp1t[...])
                    + _dot(fused_ref[...], wp1f[...]) + bp1[...])
    h = jax.nn.relu(_dot(h, wp2[...]) + bp2[...])
    out_ref[...] = _dot(h, wp3[...]) + bp3[...]


def _row_spec(cols):
    return pl.BlockSpec((_BLK, cols), lambda i: (i, 0))


def _part3d_spec():
    return pl.BlockSpec((_NC, _BLK, 32), lambda i: (0, i, 0))


def _part2d_spec():
    return pl.BlockSpec((_NC, _BLK, 1), lambda i: (0, i, 0))


def _full_spec(shape):
    nd = len(shape)
    return pl.BlockSpec(shape, lambda i: (0,) * nd)


def kernel(context, target_log, mask, spatial_ei, transit_ei, W_c1, b_c1,
           g_c, bb_c, W_c2, b_c2, mask_token, W_t, b_t, W_s1, b_s1, W_s2,
           b_s2, W_r1, b_r1, W_r2, b_r2, alpha, W_p1, b_p1, g_p, bb_p,
           W_p2, b_p2, W_p3, b_p3):
    f32 = jnp.float32
    inv = 1.0 / math.sqrt(1.0 + 1e-5)
    sc = (g_c * inv).astype(f32)
    wc1 = (W_c1 * sc[:, None]).T
    bc1 = (b_c1 * sc + bb_c)[None, :]
    sp = (g_p * inv).astype(f32)
    wp1 = W_p1 * sp[:, None]
    bp1 = (b_p1 * sp + bb_p)[None, :]
    a = jax.nn.sigmoid(alpha)
    wp1s = (a * wp1[:, :64]).T
    wp1t = ((1.0 - a) * wp1[:, :64]).T
    wp1f = wp1[:, 64:].T

    maskf = mask.astype(f32)
    es = _pack_edges(spatial_ei)
    et = _pack_edges(transit_ei)
    zeros2d = jnp.zeros((_SROWS, 32), f32)
    zeros1d = jnp.zeros((_ROWS_PER_TILE,), f32)
    ones128 = jnp.ones((_SUB,), f32)

    enc = pl.pallas_call(
        _enc_body,
        grid=(_GRID,),
        in_specs=[
            _row_spec(128), _row_spec(32), _row_spec(32),
            _full_spec((128, 64)), _full_spec((1, 64)),
            _full_spec((64, 64)), _full_spec((1, 64)),
            _full_spec((1, 32)), _full_spec((32, 32)), _full_spec((1, 32)),
            _full_spec((96, 64)), _full_spec((96, 64)),
        ],
        out_specs=[_row_spec(96)] + [_row_spec(32)] * 4,
        out_shape=[jax.ShapeDtypeStruct((N, 96), f32)]
        + [jax.ShapeDtypeStruct((N, 32), f32)] * 4,
    )
    fused, ysl, ysh, yrl, yrh = enc(
        context, target_log, maskf, wc1, bc1, W_c2.T, b_c2[None, :],
        mask_token, W_t.T, b_t[None, :], W_s1.T, W_r1.T)

    degs_p, degr_p, psl, psh, prl, prh = _prop1_call(
        es, et, ysl, ysh, yrl, yrh, zeros2d, zeros1d, ones128)
    degs_p = degs_p.reshape(_NC, _NACC, 1)
    degr_p = degr_p.reshape(_NC, _NACC, 1)
    psl, psh, prl, prh = (x.reshape(_NC, _NACC, 32)
                          for x in (psl, psh, prl, prh))

    mid = pl.pallas_call(
        _mid_body,
        grid=(_GRID,),
        in_specs=[
            _part3d_spec(), _part3d_spec(), _part3d_spec(), _part3d_spec(),
            _part2d_spec(), _part2d_spec(),
            _full_spec((1, 64)), _full_spec((64, 64)),
            _full_spec((1, 64)), _full_spec((64, 64)),
        ],
        out_specs=[_row_spec(32)] * 4,
        out_shape=[jax.ShapeDtypeStruct((N, 32), f32)] * 4,
    )
    zsl, zsh, zrl, zrh = mid(psl, psh, prl, prh, degs_p, degr_p,
                             b_s1[None, :], W_s2.T, b_r1[None, :], W_r2.T)

    qsl, qsh, qrl, qrh = (x.reshape(_NC, _NACC, 32) for x in
                          _prop2_call(es, et, zsl, zsh, zrl, zrh, zeros2d))

    head = pl.pallas_call(
        _head_body,
        grid=(_GRID,),
        in_specs=[
            _part3d_spec(), _part3d_spec(), _part3d_spec(), _part3d_spec(),
            _part2d_spec(), _part2d_spec(), _row_spec(96),
            _full_spec((1, 64)), _full_spec((1, 64)),
            _full_spec((64, 64)), _full_spec((64, 64)), _full_spec((96, 64)),
            _full_spec((1, 64)), _full_spec((64, 32)), _full_spec((1, 32)),
            _full_spec((32, 32)), _full_spec((1, 32)),
        ],
        out_specs=[_row_spec(32)],
        out_shape=[jax.ShapeDtypeStruct((N, 32), f32)],
    )
    (out,) = head(qsl, qsh, qrl, qrh, degs_p, degr_p, fused,
                  b_s2[None, :], b_r2[None, :], wp1s, wp1t, wp1f, bp1,
                  W_p2.T, b_p2[None, :], W_p3.T, b_p3[None, :])
    return out


# speedup vs baseline: 6.6392x; 1.9521x over previous
import functools
import math

import jax
import jax.numpy as jnp
from jax import lax
from jax.experimental import pallas as pl
from jax.experimental.pallas import tpu as pltpu
from jax.experimental.pallas import tpu_sc as plsc

N = 50000
E = 800000

_NC = 2
_NS = 16
_SUB = 128
_HALF_E = E // _NC
_CHUNKS_PER_TILE = -(-_HALF_E // (_SUB * _NS))
_CHUNKS_PER_CORE = _CHUNKS_PER_TILE * _NS
_PAD_E = _CHUNKS_PER_CORE * _SUB
_NTRASH = 176
_NACC = N + _NTRASH
_ROWS_PER_TILE = _NACC // _NS

_BLK = 2000
_GRID = N // _BLK


def _pack_edges(ei):
    row = ei[0].astype(jnp.int32).reshape(_NC, _HALF_E)
    col = ei[1].astype(jnp.int32).reshape(_NC, _HALF_E)
    pad = _PAD_E - _HALF_E
    trash = (N + (jnp.arange(pad, dtype=jnp.int32) % _NTRASH))[None, :]
    rowp = jnp.concatenate([row, jnp.broadcast_to(trash, (_NC, pad))], axis=1)
    colp = jnp.concatenate([col, jnp.zeros((_NC, pad), jnp.int32)], axis=1)
    packed = jnp.stack(
        [rowp.reshape(_NC, _CHUNKS_PER_CORE, _SUB),
         colp.reshape(_NC, _CHUNKS_PER_CORE, _SUB)], axis=2)
    return packed.reshape(_NC * _CHUNKS_PER_CORE, 2, _SUB)


_G = 28
_NG = _CHUNKS_PER_TILE // _G
_WCH = 32
_WROWS = _ROWS_PER_TILE // _WCH


def _combo(e_ref, y_ref, out_ref, zeros2d, dacc, acc, idx_v, rows, ones_v,
           semG, semS, semD, c, s):
    r0 = s * _ROWS_PER_TILE
    pltpu.sync_copy(zeros2d, rows[0].at[pl.ds(0, _WROWS)])
    for h in range(_WCH):
        pltpu.sync_copy(rows[0].at[pl.ds(0, _WROWS)],
                        acc.at[pl.ds(r0 + h * _WROWS, _WROWS)])
    plsc.subcore_barrier()
    base = c * _CHUNKS_PER_CORE + s * _CHUNKS_PER_TILE

    def group(g, _):
        pltpu.sync_copy(e_ref.at[pl.ds(base + g * _G, _G)], idx_v)
        gd = [None] * _G
        sd = [None] * _G
        dd = [None] * _G

        def finish(q):
            qb = q % 4
            gd[q].wait()
            sd[q] = pltpu.async_copy(rows[qb], acc.at[idx_v.at[q, 0]],
                                     semS[qb], add=True)
            if dacc is not None:
                if q >= 2:
                    dd[q - 2].wait()
                dd[q] = pltpu.async_copy(ones_v, dacc.at[idx_v.at[q, 0]],
                                         semD[q % 2], add=True)

        for j in range(_G):
            b = j % 4
            if j >= 4:
                sd[j - 4].wait()
            gd[j] = pltpu.async_copy(y_ref.at[idx_v.at[j, 1]], rows[b],
                                     semG[b])
            if j >= 3:
                finish(j - 3)
        for q in range(_G - 3, _G):
            finish(q)
        for q in range(_G - 4, _G):
            sd[q].wait()
        if dacc is not None:
            dd[_G - 2].wait()
            dd[_G - 1].wait()
        return 0

    lax.fori_loop(0, _NG, group, 0)
    plsc.subcore_barrier()
    wd = [None] * _WCH
    for h in range(_WCH):
        b = h % 4
        if h >= 4:
            wd[h - 4].wait()
        pltpu.sync_copy(acc.at[pl.ds(r0 + h * _WROWS, _WROWS)],
                        rows[b].at[pl.ds(0, _WROWS)])
        wd[h] = pltpu.async_copy(
            rows[b].at[pl.ds(0, _WROWS)],
            out_ref.at[pl.ds(c * _NACC + r0 + h * _WROWS, _WROWS)], semS[b])
    for h in range(_WCH - 4, _WCH):
        wd[h].wait()


def _deg_writeout(dacc, deg_out, dbuf, semD, c, s):
    r0 = s * _ROWS_PER_TILE
    dw = [None] * 14
    for h in range(14):
        b = h % 2
        if h >= 2:
            dw[h - 2].wait()
        pltpu.sync_copy(dacc.at[pl.ds(r0 + h * 224, 224)], dbuf.at[b])
        dw[h] = pltpu.async_copy(
            dbuf.at[b], deg_out.at[pl.ds(c * _NACC + r0 + h * 224, 224)],
            semD[b])
    dw[12].wait()
    dw[13].wait()


def _sc_prop1(es, et, ysl, ysh, yrl, yrh, zeros2d, zeros1d, ones128,
              degs_p, degr_p, psl, psh, prl, prh,
              acc, dacc_s, dacc_r, idx_v, r0v, r1v, r2v, r3v, dbuf, ones_v,
              g0, g1, g2, g3, s0, s1, s2, s3, d0, d1):
    c = lax.axis_index("c")
    s = lax.axis_index("s")
    rows = [r0v, r1v, r2v, r3v]
    semG = [g0, g1, g2, g3]
    semS = [s0, s1, s2, s3]
    semD = [d0, d1]
    pltpu.sync_copy(ones128, ones_v)
    r0 = s * _ROWS_PER_TILE
    pltpu.sync_copy(zeros1d, dbuf.at[0])
    for h in range(14):
        pltpu.sync_copy(dbuf.at[0], dacc_s.at[pl.ds(r0 + h * 224, 224)])
        pltpu.sync_copy(dbuf.at[0], dacc_r.at[pl.ds(r0 + h * 224, 224)])
    args = (acc, idx_v, rows, ones_v, semG, semS, semD, c, s)
    _combo(es, ysl, psl, zeros2d, dacc_s, *args)
    _deg_writeout(dacc_s, degs_p, dbuf, semD, c, s)
    _combo(es, ysh, psh, zeros2d, None, *args)
    _combo(et, yrl, prl, zeros2d, dacc_r, *args)
    _deg_writeout(dacc_r, degr_p, dbuf, semD, c, s)
    _combo(et, yrh, prh, zeros2d, None, *args)


def _sc_prop2(es, et, zsl, zsh, zrl, zrh, zeros2d,
              qsl, qsh, qrl, qrh,
              acc, idx_v, r0v, r1v, r2v, r3v,
              g0, g1, g2, g3, s0, s1, s2, s3):
    c = lax.axis_index("c")
    s = lax.axis_index("s")
    rows = [r0v, r1v, r2v, r3v]
    args = (acc, idx_v, rows, None, [g0, g1, g2, g3], [s0, s1, s2, s3],
            None, c, s)
    _combo(es, zsl, qsl, zeros2d, None, *args)
    _combo(es, zsh, qsh, zeros2d, None, *args)
    _combo(et, zrl, qrl, zeros2d, None, *args)
    _combo(et, zrh, qrh, zeros2d, None, *args)


_part2d = jax.ShapeDtypeStruct((_NC * _NACC,), jnp.float32)
_part3d = jax.ShapeDtypeStruct((_NC * _NACC, 32), jnp.float32)
_sc_mesh = plsc.VectorSubcoreMesh(core_axis_name="c", subcore_axis_name="s")

_sc_params = pltpu.CompilerParams(use_tc_tiling_on_sc=False)

_prop1_call = pl.kernel(
    _sc_prop1,
    out_type=(_part2d, _part2d, _part3d, _part3d, _part3d, _part3d),
    mesh=_sc_mesh,
    compiler_params=_sc_params,
    scratch_types=(
        [pltpu.VMEM_SHARED((_NACC, 32), jnp.float32),
         pltpu.VMEM_SHARED((_NACC,), jnp.float32),
         pltpu.VMEM_SHARED((_NACC,), jnp.float32),
         pltpu.VMEM((_G, 2, _SUB), jnp.int32)]
        + [pltpu.VMEM((_SUB, 32), jnp.float32)] * 4
        + [pltpu.VMEM((2, 224), jnp.float32),
           pltpu.VMEM((_SUB,), jnp.float32)]
        + [pltpu.SemaphoreType.DMA] * 10
    ),
)

_prop2_call = pl.kernel(
    _sc_prop2,
    out_type=(_part3d, _part3d, _part3d, _part3d),
    mesh=_sc_mesh,
    compiler_params=_sc_params,
    scratch_types=(
        [pltpu.VMEM_SHARED((_NACC, 32), jnp.float32),
         pltpu.VMEM((_G, 2, _SUB), jnp.int32)]
        + [pltpu.VMEM((_SUB, 32), jnp.float32)] * 4
        + [pltpu.SemaphoreType.DMA] * 8
    ),
)


def _dot(a, b):
    return jnp.dot(a, b, preferred_element_type=jnp.float32)


def _enc_body(ctx_ref, tl_ref, m_ref, wc1, bc1, wc2, bc2, mtok, wt, bt,
              ws1, wr1, fused_ref, ysl_ref, ysh_ref, yrl_ref, yrh_ref):
    h = jax.nn.relu(_dot(ctx_ref[...], wc1[...]) + bc1[...])
    ctx = jax.nn.relu(_dot(h, wc2[...]) + bc2[...])
    m = m_ref[...]
    masked = tl_ref[...] * (1.0 - m) + mtok[...] * m
    tgt = jax.nn.relu(_dot(masked, wt[...]) + bt[...])
    fused = jnp.concatenate([ctx, tgt], axis=1)
    fused_ref[...] = fused
    ys = _dot(fused, ws1[...])
    yr = _dot(fused, wr1[...])
    ysl_ref[...] = ys[:, :32]
    ysh_ref[...] = ys[:, 32:]
    yrl_ref[...] = yr[:, :32]
    yrh_ref[...] = yr[:, 32:]


def _mid_body(psl, psh, prl, prh, dsp, drp, bs1, ws2, br1, wr2,
              zsl_ref, zsh_ref, zrl_ref, zrh_ref):
    rs = 1.0 / jnp.clip(dsp[0] + dsp[1], 1.0, None)
    rr = 1.0 / jnp.clip(drp[0] + drp[1], 1.0, None)
    aggs = jnp.concatenate([psl[0] + psl[1], psh[0] + psh[1]], axis=1)
    aggr = jnp.concatenate([prl[0] + prl[1], prh[0] + prh[1]], axis=1)
    h1s = jax.nn.relu(aggs * rs + bs1[...])
    h1r = jax.nn.relu(aggr * rr + br1[...])
    zs = _dot(h1s, ws2[...])
    zr = _dot(h1r, wr2[...])
    zsl_ref[...] = zs[:, :32]
    zsh_ref[...] = zs[:, 32:]
    zrl_ref[...] = zr[:, :32]
    zrh_ref[...] = zr[:, 32:]


def _head_body(qsl, qsh, qrl, qrh, dsp, drp, fused_ref, bs2, br2,
               wp1s, wp1t, wp1f, bp1, wp2, bp2, wp3, bp3, out_ref):
    rs = 1.0 / jnp.clip(dsp[0] + dsp[1], 1.0, None)
    rr = 1.0 / jnp.clip(drp[0] + drp[1], 1.0, None)
    hs2 = jnp.concatenate([qsl[0] + qsl[1], qsh[0] + qsh[1]], axis=1) * rs + bs2[...]
    ht2 = jnp.concatenate([qrl[0] + qrl[1], qrh[0] + qrh[1]], axis=1) * rr + br2[...]
    h = jax.nn.relu(_dot(hs2, wp1s[...]) + _dot(ht2, wp1t[...])
                    + _dot(fused_ref[...], wp1f[...]) + bp1[...])
    h = jax.nn.relu(_dot(h, wp2[...]) + bp2[...])
    out_ref[...] = _dot(h, wp3[...]) + bp3[...]


def _row_spec(cols):
    return pl.BlockSpec((_BLK, cols), lambda i: (i, 0))


def _part3d_spec():
    return pl.BlockSpec((_NC, _BLK, 32), lambda i: (0, i, 0))


def _part2d_spec():
    return pl.BlockSpec((_NC, _BLK, 1), lambda i: (0, i, 0))


def _full_spec(shape):
    nd = len(shape)
    return pl.BlockSpec(shape, lambda i: (0,) * nd)


def kernel(context, target_log, mask, spatial_ei, transit_ei, W_c1, b_c1,
           g_c, bb_c, W_c2, b_c2, mask_token, W_t, b_t, W_s1, b_s1, W_s2,
           b_s2, W_r1, b_r1, W_r2, b_r2, alpha, W_p1, b_p1, g_p, bb_p,
           W_p2, b_p2, W_p3, b_p3):
    f32 = jnp.float32
    inv = 1.0 / math.sqrt(1.0 + 1e-5)
    sc = (g_c * inv).astype(f32)
    wc1 = (W_c1 * sc[:, None]).T
    bc1 = (b_c1 * sc + bb_c)[None, :]
    sp = (g_p * inv).astype(f32)
    wp1 = W_p1 * sp[:, None]
    bp1 = (b_p1 * sp + bb_p)[None, :]
    a = jax.nn.sigmoid(alpha)
    wp1s = (a * wp1[:, :64]).T
    wp1t = ((1.0 - a) * wp1[:, :64]).T
    wp1f = wp1[:, 64:].T

    maskf = mask.astype(f32)
    es = _pack_edges(spatial_ei)
    et = _pack_edges(transit_ei)
    zeros2d = jnp.zeros((_WROWS, 32), f32)
    zeros1d = jnp.zeros((224,), f32)
    ones128 = jnp.ones((_SUB,), f32)

    enc = pl.pallas_call(
        _enc_body,
        grid=(_GRID,),
        in_specs=[
            _row_spec(128), _row_spec(32), _row_spec(32),
            _full_spec((128, 64)), _full_spec((1, 64)),
            _full_spec((64, 64)), _full_spec((1, 64)),
            _full_spec((1, 32)), _full_spec((32, 32)), _full_spec((1, 32)),
            _full_spec((96, 64)), _full_spec((96, 64)),
        ],
        out_specs=[_row_spec(96)] + [_row_spec(32)] * 4,
        out_shape=[jax.ShapeDtypeStruct((N, 96), f32)]
        + [jax.ShapeDtypeStruct((N, 32), f32)] * 4,
    )
    fused, ysl, ysh, yrl, yrh = enc(
        context, target_log, maskf, wc1, bc1, W_c2.T, b_c2[None, :],
        mask_token, W_t.T, b_t[None, :], W_s1.T, W_r1.T)

    degs_p, degr_p, psl, psh, prl, prh = _prop1_call(
        es, et, ysl, ysh, yrl, yrh, zeros2d, zeros1d, ones128)
    degs_p = degs_p.reshape(_NC, _NACC, 1)
    degr_p = degr_p.reshape(_NC, _NACC, 1)
    psl, psh, prl, prh = (x.reshape(_NC, _NACC, 32)
                          for x in (psl, psh, prl, prh))

    mid = pl.pallas_call(
        _mid_body,
        grid=(_GRID,),
        in_specs=[
            _part3d_spec(), _part3d_spec(), _part3d_spec(), _part3d_spec(),
            _part2d_spec(), _part2d_spec(),
            _full_spec((1, 64)), _full_spec((64, 64)),
            _full_spec((1, 64)), _full_spec((64, 64)),
        ],
        out_specs=[_row_spec(32)] * 4,
        out_shape=[jax.ShapeDtypeStruct((N, 32), f32)] * 4,
    )
    zsl, zsh, zrl, zrh = mid(psl, psh, prl, prh, degs_p, degr_p,
                             b_s1[None, :], W_s2.T, b_r1[None, :], W_r2.T)

    qsl, qsh, qrl, qrh = (x.reshape(_NC, _NACC, 32) for x in
                          _prop2_call(es, et, zsl, zsh, zrl, zrh, zeros2d))

    head = pl.pallas_call(
        _head_body,
        grid=(_GRID,),
        in_specs=[
            _part3d_spec(), _part3d_spec(), _part3d_spec(), _part3d_spec(),
            _part2d_spec(), _part2d_spec(), _row_spec(96),
            _full_spec((1, 64)), _full_spec((1, 64)),
            _full_spec((64, 64)), _full_spec((64, 64)), _full_spec((96, 64)),
            _full_spec((1, 64)), _full_spec((64, 32)), _full_spec((1, 32)),
            _full_spec((32, 32)), _full_spec((1, 32)),
        ],
        out_specs=[_row_spec(32)],
        out_shape=[jax.ShapeDtypeStruct((N, 32), f32)],
    )
    (out,) = head(qsl, qsh, qrl, qrh, degs_p, degr_p, fused,
                  b_s2[None, :], b_r2[None, :], wp1s, wp1t, wp1f, bp1,
                  W_p2.T, b_p2[None, :], W_p3.T, b_p3[None, :])
    return out


# speedup vs baseline: 6.9325x; 1.0442x over previous
import functools
import math

import jax
import jax.numpy as jnp
from jax import lax
from jax.experimental import pallas as pl
from jax.experimental.pallas import tpu as pltpu
from jax.experimental.pallas import tpu_sc as plsc

N = 50000
E = 800000

_NC = 2
_NS = 16
_SUB = 128
_HALF_E = E // _NC
_CHUNKS_PER_TILE = -(-_HALF_E // (_SUB * _NS))
_CHUNKS_PER_CORE = _CHUNKS_PER_TILE * _NS
_PAD_E = _CHUNKS_PER_CORE * _SUB
_NTRASH = 176
_NACC = N + _NTRASH
_ROWS_PER_TILE = _NACC // _NS

_BLK = 2048
_GRID = -(-N // _BLK)


def _pack_edges(ei):
    row = ei[0].astype(jnp.int32).reshape(_NC, _HALF_E)
    col = ei[1].astype(jnp.int32).reshape(_NC, _HALF_E)
    pad = _PAD_E - _HALF_E
    trash = (N + (jnp.arange(pad, dtype=jnp.int32) % _NTRASH))[None, :]
    rowp = jnp.concatenate([row, jnp.broadcast_to(trash, (_NC, pad))], axis=1)
    colp = jnp.concatenate([col, jnp.zeros((_NC, pad), jnp.int32)], axis=1)
    return jnp.stack(
        [rowp.reshape(_NC * _CHUNKS_PER_CORE, _SUB),
         colp.reshape(_NC * _CHUNKS_PER_CORE, _SUB)], axis=0)


_G = 28
_NG = _CHUNKS_PER_TILE // _G
_WCH = 32
_WROWS = _ROWS_PER_TILE // _WCH


def _combo(e_ref, y_ref, out_ref, zeros2d, dacc, acc, idx_v, rows, ones_v,
           semG, semS, semD, c, s):
    r0 = s * _ROWS_PER_TILE
    pltpu.sync_copy(zeros2d, rows[0].at[pl.ds(0, _WROWS)])
    for h in range(_WCH):
        pltpu.sync_copy(rows[0].at[pl.ds(0, _WROWS)],
                        acc.at[pl.ds(r0 + h * _WROWS, _WROWS)])
    plsc.subcore_barrier()
    base = c * _CHUNKS_PER_CORE + s * _CHUNKS_PER_TILE

    def group(g, _):
        pltpu.sync_copy(e_ref.at[:, pl.ds(base + g * _G, _G)], idx_v)
        gd = [None] * _G
        sd = [None] * _G
        dd = [None] * _G

        def finish(q):
            qb = q % 4
            gd[q].wait()
            sd[q] = pltpu.async_copy(rows[qb], acc.at[idx_v.at[0, q]],
                                     semS[qb], add=True)
            if dacc is not None:
                if q >= 2:
                    dd[q - 2].wait()
                dd[q] = pltpu.async_copy(ones_v, dacc.at[idx_v.at[0, q]],
                                         semD[q % 2], add=True)

        for j in range(_G):
            b = j % 4
            if j >= 4:
                sd[j - 4].wait()
            gd[j] = pltpu.async_copy(y_ref.at[idx_v.at[1, j]], rows[b],
                                     semG[b])
            if j >= 3:
                finish(j - 3)
        for q in range(_G - 3, _G):
            finish(q)
        for q in range(_G - 4, _G):
            sd[q].wait()
        if dacc is not None:
            dd[_G - 2].wait()
            dd[_G - 1].wait()
        return 0

    lax.fori_loop(0, _NG, group, 0)
    plsc.subcore_barrier()
    wd = [None] * _WCH
    for h in range(_WCH):
        b = h % 4
        if h >= 4:
            wd[h - 4].wait()
        pltpu.sync_copy(acc.at[pl.ds(r0 + h * _WROWS, _WROWS)],
                        rows[b].at[pl.ds(0, _WROWS)])
        wd[h] = pltpu.async_copy(
            rows[b].at[pl.ds(0, _WROWS)],
            out_ref.at[pl.ds(c * _NACC + r0 + h * _WROWS, _WROWS)], semS[b])
    for h in range(_WCH - 4, _WCH):
        wd[h].wait()


def _deg_writeout(dacc, deg_out, dbuf, semD, c, s):
    r0 = s * _ROWS_PER_TILE
    dw = [None] * 14
    for h in range(14):
        b = h % 2
        if h >= 2:
            dw[h - 2].wait()
        pltpu.sync_copy(dacc.at[pl.ds(r0 + h * 224, 224)], dbuf.at[b])
        dw[h] = pltpu.async_copy(
            dbuf.at[b], deg_out.at[pl.ds(c * _NACC + r0 + h * 224, 224)],
            semD[b])
    dw[12].wait()
    dw[13].wait()


def _sc_prop1(es, et, ysl, ysh, yrl, yrh, zeros2d, zeros1d, ones128,
              degs_p, degr_p, psl, psh, prl, prh,
              acc, dacc_s, dacc_r, idx_v, r0v, r1v, r2v, r3v, dbuf, ones_v,
              g0, g1, g2, g3, s0, s1, s2, s3, d0, d1):
    c = lax.axis_index("c")
    s = lax.axis_index("s")
    rows = [r0v, r1v, r2v, r3v]
    semG = [g0, g1, g2, g3]
    semS = [s0, s1, s2, s3]
    semD = [d0, d1]
    pltpu.sync_copy(ones128, ones_v)
    r0 = s * _ROWS_PER_TILE
    pltpu.sync_copy(zeros1d, dbuf.at[0])
    for h in range(14):
        pltpu.sync_copy(dbuf.at[0], dacc_s.at[pl.ds(r0 + h * 224, 224)])
        pltpu.sync_copy(dbuf.at[0], dacc_r.at[pl.ds(r0 + h * 224, 224)])
    args = (acc, idx_v, rows, ones_v, semG, semS, semD, c, s)
    _combo(es, ysl, psl, zeros2d, dacc_s, *args)
    _deg_writeout(dacc_s, degs_p, dbuf, semD, c, s)
    _combo(es, ysh, psh, zeros2d, None, *args)
    _combo(et, yrl, prl, zeros2d, dacc_r, *args)
    _deg_writeout(dacc_r, degr_p, dbuf, semD, c, s)
    _combo(et, yrh, prh, zeros2d, None, *args)


def _sc_prop2(es, et, zsl, zsh, zrl, zrh, zeros2d,
              qsl, qsh, qrl, qrh,
              acc, idx_v, r0v, r1v, r2v, r3v,
              g0, g1, g2, g3, s0, s1, s2, s3):
    c = lax.axis_index("c")
    s = lax.axis_index("s")
    rows = [r0v, r1v, r2v, r3v]
    args = (acc, idx_v, rows, None, [g0, g1, g2, g3], [s0, s1, s2, s3],
            None, c, s)
    _combo(es, zsl, qsl, zeros2d, None, *args)
    _combo(es, zsh, qsh, zeros2d, None, *args)
    _combo(et, zrl, qrl, zeros2d, None, *args)
    _combo(et, zrh, qrh, zeros2d, None, *args)


_part2d = jax.ShapeDtypeStruct((_NC * _NACC,), jnp.float32)
_part3d = jax.ShapeDtypeStruct((_NC * _NACC, 32), jnp.float32)
_sc_mesh = plsc.VectorSubcoreMesh(core_axis_name="c", subcore_axis_name="s")

_sc_params = pltpu.CompilerParams(use_tc_tiling_on_sc=False)

_prop1_call = pl.kernel(
    _sc_prop1,
    out_type=(_part2d, _part2d, _part3d, _part3d, _part3d, _part3d),
    mesh=_sc_mesh,
    compiler_params=_sc_params,
    scratch_types=(
        [pltpu.VMEM_SHARED((_NACC, 32), jnp.float32),
         pltpu.VMEM_SHARED((_NACC,), jnp.float32),
         pltpu.VMEM_SHARED((_NACC,), jnp.float32),
         pltpu.VMEM((2, _G, _SUB), jnp.int32)]
        + [pltpu.VMEM((_SUB, 32), jnp.float32)] * 4
        + [pltpu.VMEM((2, 224), jnp.float32),
           pltpu.VMEM((_SUB,), jnp.float32)]
        + [pltpu.SemaphoreType.DMA] * 10
    ),
)

_prop2_call = pl.kernel(
    _sc_prop2,
    out_type=(_part3d, _part3d, _part3d, _part3d),
    mesh=_sc_mesh,
    compiler_params=_sc_params,
    scratch_types=(
        [pltpu.VMEM_SHARED((_NACC, 32), jnp.float32),
         pltpu.VMEM((2, _G, _SUB), jnp.int32)]
        + [pltpu.VMEM((_SUB, 32), jnp.float32)] * 4
        + [pltpu.SemaphoreType.DMA] * 8
    ),
)


def _dot(a, b):
    return jnp.dot(a, b, preferred_element_type=jnp.float32)


def _enc_body(ctx_ref, tlT_ref, mT_ref, wc1, bc1, wc2, bc2, mtokT, wt, btT,
              ws1, wr1, fused_ref, ysl_ref, ysh_ref, yrl_ref, yrh_ref):
    h = jax.nn.relu(_dot(ctx_ref[...], wc1[...]) + bc1[...])
    ctx = jax.nn.relu(_dot(h, wc2[...]) + bc2[...])
    mT = mT_ref[...]
    maskedT = tlT_ref[...] * (1.0 - mT) + mtokT[...] * mT
    tgtT = jax.nn.relu(_dot(wt[...], maskedT) + btT[...])
    fused = jnp.concatenate([ctx, tgtT.T], axis=1)
    fused_ref[...] = fused
    ys = _dot(fused, ws1[...])
    yr = _dot(fused, wr1[...])
    ysl_ref[...] = ys[:, :32]
    ysh_ref[...] = ys[:, 32:]
    yrl_ref[...] = yr[:, :32]
    yrh_ref[...] = yr[:, 32:]


def _mid_body(psl, psh, prl, prh, dsp, drp, bs1, ws2, br1, wr2,
              zsl_ref, zsh_ref, zrl_ref, zrh_ref):
    rs = 1.0 / jnp.clip(dsp[0] + dsp[1], 1.0, None)
    rr = 1.0 / jnp.clip(drp[0] + drp[1], 1.0, None)
    aggs = jnp.concatenate([psl[0] + psl[1], psh[0] + psh[1]], axis=1)
    aggr = jnp.concatenate([prl[0] + prl[1], prh[0] + prh[1]], axis=1)
    h1s = jax.nn.relu(aggs * rs + bs1[...])
    h1r = jax.nn.relu(aggr * rr + br1[...])
    zs = _dot(h1s, ws2[...])
    zr = _dot(h1r, wr2[...])
    zsl_ref[...] = zs[:, :32]
    zsh_ref[...] = zs[:, 32:]
    zrl_ref[...] = zr[:, :32]
    zrh_ref[...] = zr[:, 32:]


def _head_body(qsl, qsh, qrl, qrh, dsp, drp, fused_ref, bs2, br2,
               wp1s, wp1t, wp1f, bp1, wp2, bp2, wp3, bp3, out_ref):
    rs = 1.0 / jnp.clip(dsp[0] + dsp[1], 1.0, None)
    rr = 1.0 / jnp.clip(drp[0] + drp[1], 1.0, None)
    hs2 = jnp.concatenate([qsl[0] + qsl[1], qsh[0] + qsh[1]], axis=1) * rs + bs2[...]
    ht2 = jnp.concatenate([qrl[0] + qrl[1], qrh[0] + qrh[1]], axis=1) * rr + br2[...]
    h = jax.nn.relu(_dot(hs2, wp1s[...]) + _dot(ht2, wp1t[...])
                    + _dot(fused_ref[...], wp1f[...]) + bp1[...])
    h = jax.nn.relu(_dot(h, wp2[...]) + bp2[...])
    out_ref[...] = (_dot(h, wp3[...]) + bp3[...]).T


def _row_spec(cols):
    return pl.BlockSpec((_BLK, cols), lambda i: (i, 0))


def _part3d_spec():
    return pl.BlockSpec((_NC, _BLK, 32), lambda i: (0, i, 0))


def _part2d_spec():
    return pl.BlockSpec((_NC, _BLK, 1), lambda i: (0, i, 0))


def _full_spec(shape):
    nd = len(shape)
    return pl.BlockSpec(shape, lambda i: (0,) * nd)


def kernel(context, target_log, mask, spatial_ei, transit_ei, W_c1, b_c1,
           g_c, bb_c, W_c2, b_c2, mask_token, W_t, b_t, W_s1, b_s1, W_s2,
           b_s2, W_r1, b_r1, W_r2, b_r2, alpha, W_p1, b_p1, g_p, bb_p,
           W_p2, b_p2, W_p3, b_p3):
    f32 = jnp.float32
    inv = 1.0 / math.sqrt(1.0 + 1e-5)
    sc = (g_c * inv).astype(f32)
    wc1 = (W_c1 * sc[:, None]).T
    bc1 = (b_c1 * sc + bb_c)[None, :]
    sp = (g_p * inv).astype(f32)
    wp1 = W_p1 * sp[:, None]
    bp1 = (b_p1 * sp + bb_p)[None, :]
    a = jax.nn.sigmoid(alpha)
    wp1s = (a * wp1[:, :64]).T
    wp1t = ((1.0 - a) * wp1[:, :64]).T
    wp1f = wp1[:, 64:].T

    maskf = mask.astype(f32)
    es = _pack_edges(spatial_ei)
    et = _pack_edges(transit_ei)
    zeros2d = jnp.zeros((_WROWS, 32), f32)
    zeros1d = jnp.zeros((224,), f32)
    ones128 = jnp.ones((_SUB,), f32)

    enc = pl.pallas_call(
        _enc_body,
        grid=(_GRID,),
        in_specs=[
            _row_spec(128),
            pl.BlockSpec((32, _BLK), lambda i: (0, i)),
            pl.BlockSpec((32, _BLK), lambda i: (0, i)),
            _full_spec((128, 64)), _full_spec((1, 64)),
            _full_spec((64, 64)), _full_spec((1, 64)),
            _full_spec((32, 1)), _full_spec((32, 32)), _full_spec((32, 1)),
            _full_spec((96, 64)), _full_spec((96, 64)),
        ],
        out_specs=[_row_spec(96)] + [_row_spec(32)] * 4,
        out_shape=[jax.ShapeDtypeStruct((N, 96), f32)]
        + [jax.ShapeDtypeStruct((N, 32), f32)] * 4,
    )
    fused, ysl, ysh, yrl, yrh = enc(
        context, target_log.T, maskf.T, wc1, bc1, W_c2.T, b_c2[None, :],
        mask_token.T, W_t, b_t[:, None], W_s1.T, W_r1.T)

    degs_p, degr_p, psl, psh, prl, prh = _prop1_call(
        es, et, ysl, ysh, yrl, yrh, zeros2d, zeros1d, ones128)
    degs_p = degs_p.reshape(_NC, _NACC, 1)
    degr_p = degr_p.reshape(_NC, _NACC, 1)
    psl, psh, prl, prh = (x.reshape(_NC, _NACC, 32)
                          for x in (psl, psh, prl, prh))

    mid = pl.pallas_call(
        _mid_body,
        grid=(_GRID,),
        in_specs=[
            _part3d_spec(), _part3d_spec(), _part3d_spec(), _part3d_spec(),
            _part2d_spec(), _part2d_spec(),
            _full_spec((1, 64)), _full_spec((64, 64)),
            _full_spec((1, 64)), _full_spec((64, 64)),
        ],
        out_specs=[_row_spec(32)] * 4,
        out_shape=[jax.ShapeDtypeStruct((N, 32), f32)] * 4,
    )
    zsl, zsh, zrl, zrh = mid(psl, psh, prl, prh, degs_p, degr_p,
                             b_s1[None, :], W_s2.T, b_r1[None, :], W_r2.T)

    qsl, qsh, qrl, qrh = (x.reshape(_NC, _NACC, 32) for x in
                          _prop2_call(es, et, zsl, zsh, zrl, zrh, zeros2d))

    head = pl.pallas_call(
        _head_body,
        grid=(_GRID,),
        in_specs=[
            _part3d_spec(), _part3d_spec(), _part3d_spec(), _part3d_spec(),
            _part2d_spec(), _part2d_spec(), _row_spec(96),
            _full_spec((1, 64)), _full_spec((1, 64)),
            _full_spec((64, 64)), _full_spec((64, 64)), _full_spec((96, 64)),
            _full_spec((1, 64)), _full_spec((64, 32)), _full_spec((1, 32)),
            _full_spec((32, 32)), _full_spec((1, 32)),
        ],
        out_specs=[pl.BlockSpec((32, _BLK), lambda i: (0, i))],
        out_shape=[jax.ShapeDtypeStruct((32, N), f32)],
    )
    (outT,) = head(qsl, qsh, qrl, qrh, degs_p, degr_p, fused,
                   b_s2[None, :], b_r2[None, :], wp1s, wp1t, wp1f, bp1,
                   W_p2.T, b_p2[None, :], W_p3.T, b_p3[None, :])
    return outT.T


# speedup vs baseline: 7.2494x; 1.0457x over previous
import functools
import math

import jax
import jax.numpy as jnp
from jax import lax
from jax.experimental import pallas as pl
from jax.experimental.pallas import tpu as pltpu
from jax.experimental.pallas import tpu_sc as plsc

N = 50000
E = 800000

_NC = 2
_NS = 16
_SUB = 128
_HALF_E = E // _NC
_CHUNKS_PER_TILE = -(-_HALF_E // (_SUB * _NS))
_CHUNKS_PER_CORE = _CHUNKS_PER_TILE * _NS
_PAD_E = _CHUNKS_PER_CORE * _SUB
_NTRASH = 176
_NACC = N + _NTRASH
_ROWS_PER_TILE = _NACC // _NS

_BLK = 1792
_GRID = -(-N // _BLK)
_PBLK = _NACC // _BLK


def _pack_edges(ei):
    row = ei[0].astype(jnp.int32).reshape(_NC, _HALF_E)
    col = ei[1].astype(jnp.int32).reshape(_NC, _HALF_E)
    pad = _PAD_E - _HALF_E
    trash = (N + (jnp.arange(pad, dtype=jnp.int32) % _NTRASH))[None, :]
    rowp = jnp.concatenate([row, jnp.broadcast_to(trash, (_NC, pad))], axis=1)
    colp = jnp.concatenate([col, jnp.zeros((_NC, pad), jnp.int32)], axis=1)
    return jnp.stack(
        [rowp.reshape(_NC * _CHUNKS_PER_CORE, _SUB),
         colp.reshape(_NC * _CHUNKS_PER_CORE, _SUB)], axis=0)


_G = 28
_NG = _CHUNKS_PER_TILE // _G
_WCH = 32
_WROWS = _ROWS_PER_TILE // _WCH


def _combo(e_ref, y_ref, out_ref, zeros2d, dacc, acc, idx_v, rows, ones_v,
           semG, semS, semD, c, s):
    r0 = s * _ROWS_PER_TILE
    pltpu.sync_copy(zeros2d, rows[0].at[pl.ds(0, _WROWS)])
    for h in range(_WCH):
        pltpu.sync_copy(rows[0].at[pl.ds(0, _WROWS)],
                        acc.at[pl.ds(r0 + h * _WROWS, _WROWS)])
    plsc.subcore_barrier()
    base = c * _CHUNKS_PER_CORE + s * _CHUNKS_PER_TILE

    def group(g, _):
        pltpu.sync_copy(e_ref.at[:, pl.ds(base + g * _G, _G)], idx_v)
        gd = [None] * _G
        sd = [None] * _G
        dd = [None] * _G

        def finish(q):
            qb = q % 4
            gd[q].wait()
            sd[q] = pltpu.async_copy(rows[qb], acc.at[idx_v.at[0, q]],
                                     semS[qb], add=True)
            if dacc is not None:
                if q >= 2:
                    dd[q - 2].wait()
                dd[q] = pltpu.async_copy(ones_v, dacc.at[idx_v.at[0, q]],
                                         semD[q % 2], add=True)

        for j in range(_G):
            b = j % 4
            if j >= 4:
                sd[j - 4].wait()
            gd[j] = pltpu.async_copy(y_ref.at[idx_v.at[1, j]], rows[b],
                                     semG[b])
            if j >= 3:
                finish(j - 3)
        for q in range(_G - 3, _G):
            finish(q)
        for q in range(_G - 4, _G):
            sd[q].wait()
        if dacc is not None:
            dd[_G - 2].wait()
            dd[_G - 1].wait()
        return 0

    lax.fori_loop(0, _NG, group, 0)
    plsc.subcore_barrier()
    wd = [None] * _WCH
    for h in range(_WCH):
        b = h % 4
        if h >= 4:
            wd[h - 4].wait()
        pltpu.sync_copy(acc.at[pl.ds(r0 + h * _WROWS, _WROWS)],
                        rows[b].at[pl.ds(0, _WROWS)])
        wd[h] = pltpu.async_copy(
            rows[b].at[pl.ds(0, _WROWS)],
            out_ref.at[pl.ds(c * _NACC + r0 + h * _WROWS, _WROWS)], semS[b])
    for h in range(_WCH - 4, _WCH):
        wd[h].wait()


def _deg_writeout(dacc, deg_out, dbuf, semD, c, s):
    r0 = s * _ROWS_PER_TILE
    dw = [None] * 14
    for h in range(14):
        b = h % 2
        if h >= 2:
            dw[h - 2].wait()
        pltpu.sync_copy(dacc.at[pl.ds(r0 + h * 224, 224)], dbuf.at[b])
        dw[h] = pltpu.async_copy(
            dbuf.at[b], deg_out.at[pl.ds(c * _NACC + r0 + h * 224, 224)],
            semD[b])
    dw[12].wait()
    dw[13].wait()


def _sc_prop1(es, et, ysl, ysh, yrl, yrh, zeros2d, zeros1d, ones128,
              degs_p, degr_p, psl, psh, prl, prh,
              acc, dacc_s, dacc_r, idx_v, r0v, r1v, r2v, r3v, dbuf, ones_v,
              g0, g1, g2, g3, s0, s1, s2, s3, d0, d1):
    c = lax.axis_index("c")
    s = lax.axis_index("s")
    rows = [r0v, r1v, r2v, r3v]
    semG = [g0, g1, g2, g3]
    semS = [s0, s1, s2, s3]
    semD = [d0, d1]
    pltpu.sync_copy(ones128, ones_v)
    r0 = s * _ROWS_PER_TILE
    pltpu.sync_copy(zeros1d, dbuf.at[0])
    for h in range(14):
        pltpu.sync_copy(dbuf.at[0], dacc_s.at[pl.ds(r0 + h * 224, 224)])
        pltpu.sync_copy(dbuf.at[0], dacc_r.at[pl.ds(r0 + h * 224, 224)])
    args = (acc, idx_v, rows, ones_v, semG, semS, semD, c, s)
    _combo(es, ysl, psl, zeros2d, dacc_s, *args)
    _deg_writeout(dacc_s, degs_p, dbuf, semD, c, s)
    _combo(es, ysh, psh, zeros2d, None, *args)
    _combo(et, yrl, prl, zeros2d, dacc_r, *args)
    _deg_writeout(dacc_r, degr_p, dbuf, semD, c, s)
    _combo(et, yrh, prh, zeros2d, None, *args)


def _sc_prop2(es, et, zsl, zsh, zrl, zrh, zeros2d,
              qsl, qsh, qrl, qrh,
              acc, idx_v, r0v, r1v, r2v, r3v,
              g0, g1, g2, g3, s0, s1, s2, s3):
    c = lax.axis_index("c")
    s = lax.axis_index("s")
    rows = [r0v, r1v, r2v, r3v]
    args = (acc, idx_v, rows, None, [g0, g1, g2, g3], [s0, s1, s2, s3],
            None, c, s)
    _combo(es, zsl, qsl, zeros2d, None, *args)
    _combo(es, zsh, qsh, zeros2d, None, *args)
    _combo(et, zrl, qrl, zeros2d, None, *args)
    _combo(et, zrh, qrh, zeros2d, None, *args)


_part2d = jax.ShapeDtypeStruct((_NC * _NACC,), jnp.float32)
_part3d = jax.ShapeDtypeStruct((_NC * _NACC, 32), jnp.float32)
_sc_mesh = plsc.VectorSubcoreMesh(core_axis_name="c", subcore_axis_name="s")

_sc_params = pltpu.CompilerParams(use_tc_tiling_on_sc=False)

_prop1_call = pl.kernel(
    _sc_prop1,
    out_type=(_part2d, _part2d, _part3d, _part3d, _part3d, _part3d),
    mesh=_sc_mesh,
    compiler_params=_sc_params,
    scratch_types=(
        [pltpu.VMEM_SHARED((_NACC, 32), jnp.float32),
         pltpu.VMEM_SHARED((_NACC,), jnp.float32),
         pltpu.VMEM_SHARED((_NACC,), jnp.float32),
         pltpu.VMEM((2, _G, _SUB), jnp.int32)]
        + [pltpu.VMEM((_SUB, 32), jnp.float32)] * 4
        + [pltpu.VMEM((2, 224), jnp.float32),
           pltpu.VMEM((_SUB,), jnp.float32)]
        + [pltpu.SemaphoreType.DMA] * 10
    ),
)

_prop2_call = pl.kernel(
    _sc_prop2,
    out_type=(_part3d, _part3d, _part3d, _part3d),
    mesh=_sc_mesh,
    compiler_params=_sc_params,
    scratch_types=(
        [pltpu.VMEM_SHARED((_NACC, 32), jnp.float32),
         pltpu.VMEM((2, _G, _SUB), jnp.int32)]
        + [pltpu.VMEM((_SUB, 32), jnp.float32)] * 4
        + [pltpu.SemaphoreType.DMA] * 8
    ),
)


def _dot(a, b):
    return jnp.dot(a, b, preferred_element_type=jnp.float32)


def _enc_body(ctx_ref, tlT_ref, mT_ref, wc1, bc1, wc2, bc2, mtokT, wt, btT,
              ws1, wr1, fused_ref, ysl_ref, ysh_ref, yrl_ref, yrh_ref):
    h = jax.nn.relu(_dot(ctx_ref[...], wc1[...]) + bc1[...])
    ctx = jax.nn.relu(_dot(h, wc2[...]) + bc2[...])
    mT = mT_ref[...]
    maskedT = tlT_ref[...] * (1.0 - mT) + mtokT[...] * mT
    tgtT = jax.nn.relu(_dot(wt[...], maskedT) + btT[...])
    fused = jnp.concatenate([ctx, tgtT.T], axis=1)
    fused_ref[...] = fused
    ys = _dot(fused, ws1[...])
    yr = _dot(fused, wr1[...])
    ysl_ref[...] = ys[:, :32]
    ysh_ref[...] = ys[:, 32:]
    yrl_ref[...] = yr[:, :32]
    yrh_ref[...] = yr[:, 32:]


def _mid_body(psl0, psl1, psh0, psh1, prl0, prl1, prh0, prh1, dsb, drb,
              bs1, ws2, br1, wr2, zsl_ref, zsh_ref, zrl_ref, zrh_ref):
    rs = 1.0 / jnp.clip(dsb[...], 1.0, None)
    rr = 1.0 / jnp.clip(drb[...], 1.0, None)
    aggs = jnp.concatenate([(psl0[...] + psl1[...]) * rs,
                            (psh0[...] + psh1[...]) * rs], axis=1)
    aggr = jnp.concatenate([(prl0[...] + prl1[...]) * rr,
                            (prh0[...] + prh1[...]) * rr], axis=1)
    h1s = jax.nn.relu(aggs + bs1[...])
    h1r = jax.nn.relu(aggr + br1[...])
    zs = _dot(h1s, ws2[...])
    zr = _dot(h1r, wr2[...])
    zsl_ref[...] = zs[:, :32]
    zsh_ref[...] = zs[:, 32:]
    zrl_ref[...] = zr[:, :32]
    zrh_ref[...] = zr[:, 32:]


def _head_body(qsl0, qsl1, qsh0, qsh1, qrl0, qrl1, qrh0, qrh1, dsb, drb,
               fused_ref, bs2, br2, wp1s, wp1t, wp1f, bp1, wp2, bp2, wp3,
               bp3, out_ref):
    rs = 1.0 / jnp.clip(dsb[...], 1.0, None)
    rr = 1.0 / jnp.clip(drb[...], 1.0, None)
    hs2 = jnp.concatenate([(qsl0[...] + qsl1[...]) * rs,
                           (qsh0[...] + qsh1[...]) * rs], axis=1) + bs2[...]
    ht2 = jnp.concatenate([(qrl0[...] + qrl1[...]) * rr,
                           (qrh0[...] + qrh1[...]) * rr], axis=1) + br2[...]
    h = jax.nn.relu(_dot(hs2, wp1s[...]) + _dot(ht2, wp1t[...])
                    + _dot(fused_ref[...], wp1f[...]) + bp1[...])
    h = jax.nn.relu(_dot(h, wp2[...]) + bp2[...])
    out_ref[...] = (_dot(h, wp3[...]) + bp3[...]).T


def _row_spec(cols):
    return pl.BlockSpec((_BLK, cols), lambda i: (i, 0))


def _p0_spec():
    return pl.BlockSpec((_BLK, 32), lambda i: (i, 0))


def _p1_spec():
    return pl.BlockSpec((_BLK, 32), lambda i: (i + _PBLK, 0))


def _full_spec(shape):
    nd = len(shape)
    return pl.BlockSpec(shape, lambda i: (0,) * nd)


def kernel(context, target_log, mask, spatial_ei, transit_ei, W_c1, b_c1,
           g_c, bb_c, W_c2, b_c2, mask_token, W_t, b_t, W_s1, b_s1, W_s2,
           b_s2, W_r1, b_r1, W_r2, b_r2, alpha, W_p1, b_p1, g_p, bb_p,
           W_p2, b_p2, W_p3, b_p3):
    f32 = jnp.float32
    inv = 1.0 / math.sqrt(1.0 + 1e-5)
    sc = (g_c * inv).astype(f32)
    wc1 = (W_c1 * sc[:, None]).T
    bc1 = (b_c1 * sc + bb_c)[None, :]
    sp = (g_p * inv).astype(f32)
    wp1 = W_p1 * sp[:, None]
    bp1 = (b_p1 * sp + bb_p)[None, :]
    a = jax.nn.sigmoid(alpha)
    wp1s = (a * wp1[:, :64]).T
    wp1t = ((1.0 - a) * wp1[:, :64]).T
    wp1f = wp1[:, 64:].T

    maskf = mask.astype(f32)
    es = _pack_edges(spatial_ei)
    et = _pack_edges(transit_ei)
    zeros2d = jnp.zeros((_WROWS, 32), f32)
    zeros1d = jnp.zeros((224,), f32)
    ones128 = jnp.ones((_SUB,), f32)

    enc = pl.pallas_call(
        _enc_body,
        grid=(_GRID,),
        in_specs=[
            _row_spec(128),
            pl.BlockSpec((32, _BLK), lambda i: (0, i)),
            pl.BlockSpec((32, _BLK), lambda i: (0, i)),
            _full_spec((128, 64)), _full_spec((1, 64)),
            _full_spec((64, 64)), _full_spec((1, 64)),
            _full_spec((32, 1)), _full_spec((32, 32)), _full_spec((32, 1)),
            _full_spec((96, 64)), _full_spec((96, 64)),
        ],
        out_specs=[_row_spec(96)] + [_row_spec(32)] * 4,
        out_shape=[jax.ShapeDtypeStruct((N, 96), f32)]
        + [jax.ShapeDtypeStruct((N, 32), f32)] * 4,
    )
    fused, ysl, ysh, yrl, yrh = enc(
        context, target_log.T, maskf.T, wc1, bc1, W_c2.T, b_c2[None, :],
        mask_token.T, W_t, b_t[:, None], W_s1.T, W_r1.T)

    degs_p, degr_p, psl, psh, prl, prh = _prop1_call(
        es, et, ysl, ysh, yrl, yrh, zeros2d, zeros1d, ones128)
    dsum_s = degs_p.reshape(_NC, _NACC)
    dsum_r = degr_p.reshape(_NC, _NACC)
    degb_s = jnp.broadcast_to((dsum_s[0] + dsum_s[1])[:, None], (_NACC, 32))
    degb_r = jnp.broadcast_to((dsum_r[0] + dsum_r[1])[:, None], (_NACC, 32))

    mid = pl.pallas_call(
        _mid_body,
        grid=(_GRID,),
        in_specs=[
            _p0_spec(), _p1_spec(), _p0_spec(), _p1_spec(),
            _p0_spec(), _p1_spec(), _p0_spec(), _p1_spec(),
            pl.BlockSpec((_BLK, 32), lambda i: (i, 0)),
            pl.BlockSpec((_BLK, 32), lambda i: (i, 0)),
            _full_spec((1, 64)), _full_spec((64, 64)),
            _full_spec((1, 64)), _full_spec((64, 64)),
        ],
        out_specs=[_row_spec(32)] * 4,
        out_shape=[jax.ShapeDtypeStruct((N, 32), f32)] * 4,
    )
    zsl, zsh, zrl, zrh = mid(psl, psl, psh, psh, prl, prl, prh, prh,
                             degb_s, degb_r,
                             b_s1[None, :], W_s2.T, b_r1[None, :], W_r2.T)

    qsl, qsh, qrl, qrh = _prop2_call(es, et, zsl, zsh, zrl, zrh, zeros2d)

    head = pl.pallas_call(
        _head_body,
        grid=(_GRID,),
        in_specs=[
            _p0_spec(), _p1_spec(), _p0_spec(), _p1_spec(),
            _p0_spec(), _p1_spec(), _p0_spec(), _p1_spec(),
            pl.BlockSpec((_BLK, 32), lambda i: (i, 0)),
            pl.BlockSpec((_BLK, 32), lambda i: (i, 0)),
            _row_spec(96),
            _full_spec((1, 64)), _full_spec((1, 64)),
            _full_spec((64, 64)), _full_spec((64, 64)), _full_spec((96, 64)),
            _full_spec((1, 64)), _full_spec((64, 32)), _full_spec((1, 32)),
            _full_spec((32, 32)), _full_spec((1, 32)),
        ],
        out_specs=[pl.BlockSpec((32, _BLK), lambda i: (0, i))],
        out_shape=[jax.ShapeDtypeStruct((32, N), f32)],
    )
    (outT,) = head(qsl, qsl, qsh, qsh, qrl, qrl, qrh, qrh, degb_s, degb_r,
                   fused, b_s2[None, :], b_r2[None, :], wp1s, wp1t, wp1f,
                   bp1, W_p2.T, b_p2[None, :], W_p3.T, b_p3[None, :])
    return outT.T


# speedup vs baseline: 8.6128x; 1.1881x over previous
import functools
import math

import jax
import jax.numpy as jnp
from jax import lax
from jax.experimental import pallas as pl
from jax.experimental.pallas import tpu as pltpu
from jax.experimental.pallas import tpu_sc as plsc

N = 50000
E = 800000

_NC = 2
_NS = 16
_SUB = 128
_HALF_E = E // _NC
_CHUNKS_PER_TILE = -(-_HALF_E // (_SUB * _NS))
_CHUNKS_PER_CORE = _CHUNKS_PER_TILE * _NS
_PAD_E = _CHUNKS_PER_CORE * _SUB
_NTRASH = 176
_NACC = N + _NTRASH
_ROWS_PER_TILE = _NACC // _NS

_BLK = 1792
_GRID = -(-N // _BLK)
_PBLK = _NACC // _BLK


def _pack_edges(ei):
    pad = _PAD_E - _HALF_E
    trash = (N + (jnp.arange(pad, dtype=jnp.int32) % _NTRASH))[None, None, :]
    eip = jnp.concatenate(
        [ei.astype(jnp.int32).reshape(2, _NC, _HALF_E),
         jnp.broadcast_to(trash, (2, _NC, pad))], axis=2)
    return eip.reshape(2, _NC * _CHUNKS_PER_CORE, _SUB)


_G = 28
_NG = _CHUNKS_PER_TILE // _G
_WCH = 32
_WROWS = _ROWS_PER_TILE // _WCH


def _combo(e_ref, y_ref, out_ref, zeros2d, dacc, acc, idx_v, rows, ones_v,
           semG, semS, semD, c, s):
    r0 = s * _ROWS_PER_TILE
    pltpu.sync_copy(zeros2d, rows[0].at[pl.ds(0, _WROWS)])
    for h in range(_WCH):
        pltpu.sync_copy(rows[0].at[pl.ds(0, _WROWS)],
                        acc.at[pl.ds(r0 + h * _WROWS, _WROWS)])
    plsc.subcore_barrier()
    base = c * _CHUNKS_PER_CORE + s * _CHUNKS_PER_TILE

    def group(g, _):
        pltpu.sync_copy(e_ref.at[:, pl.ds(base + g * _G, _G)], idx_v)
        gd = [None] * _G
        sd = [None] * _G
        dd = [None] * _G

        def finish(q):
            qb = q % 4
            gd[q].wait()
            sd[q] = pltpu.async_copy(rows[qb], acc.at[idx_v.at[0, q]],
                                     semS[qb], add=True)
            if dacc is not None:
                if q >= 2:
                    dd[q - 2].wait()
                dd[q] = pltpu.async_copy(ones_v, dacc.at[idx_v.at[0, q]],
                                         semD[q % 2], add=True)

        for j in range(_G):
            b = j % 4
            if j >= 4:
                sd[j - 4].wait()
            gd[j] = pltpu.async_copy(y_ref.at[idx_v.at[1, j]], rows[b],
                                     semG[b])
            if j >= 3:
                finish(j - 3)
        for q in range(_G - 3, _G):
            finish(q)
        for q in range(_G - 4, _G):
            sd[q].wait()
        if dacc is not None:
            dd[_G - 2].wait()
            dd[_G - 1].wait()
        return 0

    lax.fori_loop(0, _NG, group, 0)
    plsc.subcore_barrier()
    wd = [None] * _WCH
    for h in range(_WCH):
        b = h % 4
        if h >= 4:
            wd[h - 4].wait()
        pltpu.sync_copy(acc.at[pl.ds(r0 + h * _WROWS, _WROWS)],
                        rows[b].at[pl.ds(0, _WROWS)])
        wd[h] = pltpu.async_copy(
            rows[b].at[pl.ds(0, _WROWS)],
            out_ref.at[pl.ds(c * _NACC + r0 + h * _WROWS, _WROWS)], semS[b])
    for h in range(_WCH - 4, _WCH):
        wd[h].wait()


def _deg_writeout(dacc, deg_out, dbuf, semD, c, s):
    r0 = s * _ROWS_PER_TILE
    dw = [None] * 14
    for h in range(14):
        b = h % 2
        if h >= 2:
            dw[h - 2].wait()
        pltpu.sync_copy(dacc.at[pl.ds(r0 + h * 224, 224)], dbuf.at[b])
        dw[h] = pltpu.async_copy(
            dbuf.at[b], deg_out.at[pl.ds(c * _NACC + r0 + h * 224, 224)],
            semD[b])
    dw[12].wait()
    dw[13].wait()


def _sc_prop1(es, et, ysl, ysh, yrl, yrh, zeros2d, zeros1d, ones128,
              degs_p, degr_p, psl, psh, prl, prh,
              acc, dacc_s, dacc_r, idx_v, r0v, r1v, r2v, r3v, dbuf, ones_v,
              g0, g1, g2, g3, s0, s1, s2, s3, d0, d1):
    c = lax.axis_index("c")
    s = lax.axis_index("s")
    rows = [r0v, r1v, r2v, r3v]
    semG = [g0, g1, g2, g3]
    semS = [s0, s1, s2, s3]
    semD = [d0, d1]
    pltpu.sync_copy(ones128, ones_v)
    r0 = s * _ROWS_PER_TILE
    pltpu.sync_copy(zeros1d, dbuf.at[0])
    for h in range(14):
        pltpu.sync_copy(dbuf.at[0], dacc_s.at[pl.ds(r0 + h * 224, 224)])
        pltpu.sync_copy(dbuf.at[0], dacc_r.at[pl.ds(r0 + h * 224, 224)])
    args = (acc, idx_v, rows, ones_v, semG, semS, semD, c, s)
    _combo(es, ysl, psl, zeros2d, dacc_s, *args)
    _deg_writeout(dacc_s, degs_p, dbuf, semD, c, s)
    _combo(es, ysh, psh, zeros2d, None, *args)
    _combo(et, yrl, prl, zeros2d, dacc_r, *args)
    _deg_writeout(dacc_r, degr_p, dbuf, semD, c, s)
    _combo(et, yrh, prh, zeros2d, None, *args)


def _sc_prop2(es, et, zsl, zsh, zrl, zrh, zeros2d,
              qsl, qsh, qrl, qrh,
              acc, idx_v, r0v, r1v, r2v, r3v,
              g0, g1, g2, g3, s0, s1, s2, s3):
    c = lax.axis_index("c")
    s = lax.axis_index("s")
    rows = [r0v, r1v, r2v, r3v]
    args = (acc, idx_v, rows, None, [g0, g1, g2, g3], [s0, s1, s2, s3],
            None, c, s)
    _combo(es, zsl, qsl, zeros2d, None, *args)
    _combo(es, zsh, qsh, zeros2d, None, *args)
    _combo(et, zrl, qrl, zeros2d, None, *args)
    _combo(et, zrh, qrh, zeros2d, None, *args)


_part2d = jax.ShapeDtypeStruct((_NC * _NACC,), jnp.float32)
_part3d = jax.ShapeDtypeStruct((_NC * _NACC, 32), jnp.float32)
_sc_mesh = plsc.VectorSubcoreMesh(core_axis_name="c", subcore_axis_name="s")

_sc_params = pltpu.CompilerParams(use_tc_tiling_on_sc=False)

_prop1_call = pl.kernel(
    _sc_prop1,
    out_type=(_part2d, _part2d, _part3d, _part3d, _part3d, _part3d),
    mesh=_sc_mesh,
    compiler_params=_sc_params,
    scratch_types=(
        [pltpu.VMEM_SHARED((_NACC, 32), jnp.float32),
         pltpu.VMEM_SHARED((_NACC,), jnp.float32),
         pltpu.VMEM_SHARED((_NACC,), jnp.float32),
         pltpu.VMEM((2, _G, _SUB), jnp.int32)]
        + [pltpu.VMEM((_SUB, 32), jnp.float32)] * 4
        + [pltpu.VMEM((2, 224), jnp.float32),
           pltpu.VMEM((_SUB,), jnp.float32)]
        + [pltpu.SemaphoreType.DMA] * 10
    ),
)

_prop2_call = pl.kernel(
    _sc_prop2,
    out_type=(_part3d, _part3d, _part3d, _part3d),
    mesh=_sc_mesh,
    compiler_params=_sc_params,
    scratch_types=(
        [pltpu.VMEM_SHARED((_NACC, 32), jnp.float32),
         pltpu.VMEM((2, _G, _SUB), jnp.int32)]
        + [pltpu.VMEM((_SUB, 32), jnp.float32)] * 4
        + [pltpu.SemaphoreType.DMA] * 8
    ),
)


def _dot(a, b):
    return jnp.dot(a, b, preferred_element_type=jnp.float32)


def _enc_body(ctx_ref, tlT_ref, mT_ref, wc1, bc1, wc2, bc2, mtokT, wt, btT,
              ws1, wr1, fused_ref, ysl_ref, ysh_ref, yrl_ref, yrh_ref):
    h = jax.nn.relu(_dot(ctx_ref[...], wc1[...]) + bc1[...])
    ctx = jax.nn.relu(_dot(h, wc2[...]) + bc2[...])
    mT = mT_ref[...]
    maskedT = tlT_ref[...] * (1.0 - mT) + mtokT[...] * mT
    tgtT = jax.nn.relu(_dot(wt[...], maskedT) + btT[...])
    fused = jnp.concatenate([ctx, tgtT.T], axis=1)
    fused_ref[...] = fused
    ys = _dot(fused, ws1[...])
    yr = _dot(fused, wr1[...])
    ysl_ref[...] = ys[:, :32]
    ysh_ref[...] = ys[:, 32:]
    yrl_ref[...] = yr[:, :32]
    yrh_ref[...] = yr[:, 32:]


def _mid_body(psl0, psl1, psh0, psh1, prl0, prl1, prh0, prh1, dsb, drb,
              bs1, ws2, br1, wr2, zsl_ref, zsh_ref, zrl_ref, zrh_ref):
    rs = 1.0 / jnp.clip(dsb[...], 1.0, None)
    rr = 1.0 / jnp.clip(drb[...], 1.0, None)
    aggs = jnp.concatenate([(psl0[...] + psl1[...]) * rs,
                            (psh0[...] + psh1[...]) * rs], axis=1)
    aggr = jnp.concatenate([(prl0[...] + prl1[...]) * rr,
                            (prh0[...] + prh1[...]) * rr], axis=1)
    h1s = jax.nn.relu(aggs + bs1[...])
    h1r = jax.nn.relu(aggr + br1[...])
    zs = _dot(h1s, ws2[...])
    zr = _dot(h1r, wr2[...])
    zsl_ref[...] = zs[:, :32]
    zsh_ref[...] = zs[:, 32:]
    zrl_ref[...] = zr[:, :32]
    zrh_ref[...] = zr[:, 32:]


def _head_body(qsl0, qsl1, qsh0, qsh1, qrl0, qrl1, qrh0, qrh1, dsb, drb,
               fused_ref, bs2, br2, wp1s, wp1t, wp1f, bp1, wp2, bp2, wp3,
               bp3, out_ref):
    rs = 1.0 / jnp.clip(dsb[...], 1.0, None)
    rr = 1.0 / jnp.clip(drb[...], 1.0, None)
    hs2 = jnp.concatenate([(qsl0[...] + qsl1[...]) * rs,
                           (qsh0[...] + qsh1[...]) * rs], axis=1) + bs2[...]
    ht2 = jnp.concatenate([(qrl0[...] + qrl1[...]) * rr,
                           (qrh0[...] + qrh1[...]) * rr], axis=1) + br2[...]
    h = jax.nn.relu(_dot(hs2, wp1s[...]) + _dot(ht2, wp1t[...])
                    + _dot(fused_ref[...], wp1f[...]) + bp1[...])
    h = jax.nn.relu(_dot(h, wp2[...]) + bp2[...])
    out_ref[...] = (_dot(h, wp3[...]) + bp3[...]).T


def _row_spec(cols):
    return pl.BlockSpec((_BLK, cols), lambda i: (i, 0))


def _p0_spec():
    return pl.BlockSpec((_BLK, 32), lambda i: (i, 0))


def _p1_spec():
    return pl.BlockSpec((_BLK, 32), lambda i: (i + _PBLK, 0))


def _full_spec(shape):
    nd = len(shape)
    return pl.BlockSpec(shape, lambda i: (0,) * nd)


def kernel(context, target_log, mask, spatial_ei, transit_ei, W_c1, b_c1,
           g_c, bb_c, W_c2, b_c2, mask_token, W_t, b_t, W_s1, b_s1, W_s2,
           b_s2, W_r1, b_r1, W_r2, b_r2, alpha, W_p1, b_p1, g_p, bb_p,
           W_p2, b_p2, W_p3, b_p3):
    f32 = jnp.float32
    inv = 1.0 / math.sqrt(1.0 + 1e-5)
    sc = (g_c * inv).astype(f32)
    wc1 = (W_c1 * sc[:, None]).T
    bc1 = (b_c1 * sc + bb_c)[None, :]
    sp = (g_p * inv).astype(f32)
    wp1 = W_p1 * sp[:, None]
    bp1 = (b_p1 * sp + bb_p)[None, :]
    a = jax.nn.sigmoid(alpha)
    wp1s = (a * wp1[:, :64]).T
    wp1t = ((1.0 - a) * wp1[:, :64]).T
    wp1f = wp1[:, 64:].T

    maskf = mask.astype(f32)
    es = _pack_edges(spatial_ei)
    et = _pack_edges(transit_ei)
    zeros2d = jnp.zeros((_WROWS, 32), f32)
    zeros1d = jnp.zeros((224,), f32)
    ones128 = jnp.ones((_SUB,), f32)

    enc = pl.pallas_call(
        _enc_body,
        grid=(_GRID,),
        in_specs=[
            _row_spec(128),
            pl.BlockSpec((32, _BLK), lambda i: (0, i)),
            pl.BlockSpec((32, _BLK), lambda i: (0, i)),
            _full_spec((128, 64)), _full_spec((1, 64)),
            _full_spec((64, 64)), _full_spec((1, 64)),
            _full_spec((32, 1)), _full_spec((32, 32)), _full_spec((32, 1)),
            _full_spec((96, 64)), _full_spec((96, 64)),
        ],
        out_specs=[_row_spec(96)] + [_row_spec(32)] * 4,
        out_shape=[jax.ShapeDtypeStruct((N, 96), f32)]
        + [jax.ShapeDtypeStruct((_NACC, 32), f32)] * 4,
    )
    fused, ysl, ysh, yrl, yrh = enc(
        context, target_log.T, maskf.T, wc1, bc1, W_c2.T, b_c2[None, :],
        mask_token.T, W_t, b_t[:, None], W_s1.T, W_r1.T)

    degs_p, degr_p, psl, psh, prl, prh = _prop1_call(
        es, et, ysl, ysh, yrl, yrh, zeros2d, zeros1d, ones128)
    dsum_s = degs_p.reshape(_NC, _NACC)
    dsum_r = degr_p.reshape(_NC, _NACC)
    degb_s = jnp.broadcast_to((dsum_s[0] + dsum_s[1])[:, None], (_NACC, 32))
    degb_r = jnp.broadcast_to((dsum_r[0] + dsum_r[1])[:, None], (_NACC, 32))

    mid = pl.pallas_call(
        _mid_body,
        grid=(_GRID,),
        in_specs=[
            _p0_spec(), _p1_spec(), _p0_spec(), _p1_spec(),
            _p0_spec(), _p1_spec(), _p0_spec(), _p1_spec(),
            pl.BlockSpec((_BLK, 32), lambda i: (i, 0)),
            pl.BlockSpec((_BLK, 32), lambda i: (i, 0)),
            _full_spec((1, 64)), _full_spec((64, 64)),
            _full_spec((1, 64)), _full_spec((64, 64)),
        ],
        out_specs=[_row_spec(32)] * 4,
        out_shape=[jax.ShapeDtypeStruct((_NACC, 32), f32)] * 4,
    )
    zsl, zsh, zrl, zrh = mid(psl, psl, psh, psh, prl, prl, prh, prh,
                             degb_s, degb_r,
                             b_s1[None, :], W_s2.T, b_r1[None, :], W_r2.T)

    qsl, qsh, qrl, qrh = _prop2_call(es, et, zsl, zsh, zrl, zrh, zeros2d)

    head = pl.pallas_call(
        _head_body,
        grid=(_GRID,),
        in_specs=[
            _p0_spec(), _p1_spec(), _p0_spec(), _p1_spec(),
            _p0_spec(), _p1_spec(), _p0_spec(), _p1_spec(),
            pl.BlockSpec((_BLK, 32), lambda i: (i, 0)),
            pl.BlockSpec((_BLK, 32), lambda i: (i, 0)),
            _row_spec(96),
            _full_spec((1, 64)), _full_spec((1, 64)),
            _full_spec((64, 64)), _full_spec((64, 64)), _full_spec((96, 64)),
            _full_spec((1, 64)), _full_spec((64, 32)), _full_spec((1, 32)),
            _full_spec((32, 32)), _full_spec((1, 32)),
        ],
        out_specs=[pl.BlockSpec((32, _BLK), lambda i: (0, i))],
        out_shape=[jax.ShapeDtypeStruct((32, N), f32)],
    )
    (outT,) = head(qsl, qsl, qsh, qsh, qrl, qrl, qrh, qrh, degb_s, degb_r,
                   fused, b_s2[None, :], b_r2[None, :], wp1s, wp1t, wp1f,
                   bp1, W_p2.T, b_p2[None, :], W_p3.T, b_p3[None, :])
    return outT.T
